# Initial kernel scaffold; baseline (speedup 1.0000x reference)
import jax
import jax.numpy as jnp
from jax.experimental import pallas as pl


def kernel(x_coord, pndata, latent_tokens_coord, W_lift, b_lift, W1, b1, W2, b2):
    raise NotImplementedError("write your pallas kernel here")


# speedup vs baseline: 3.2140x; 3.2140x over previous
import functools

import jax
import jax.numpy as jnp
from jax.experimental import pallas as pl
from jax.experimental.pallas import tpu as pltpu

_RADIUS = 0.09
_R2 = _RADIUS * _RADIUS


def _tc_body(xq_ref, xqT_ref, yT_ref, pn_ref, Wl_ref, bl_ref, W1a_ref, W1b_ref,
             b1_ref, W2_ref, b2_ref, o_ref, num_ref, cnt_ref, *, nblocks, h_dim,
             cd):
    i = pl.program_id(0)

    @pl.when(i == 0)
    def _init():
        num_ref[...] = jnp.zeros_like(num_ref)
        cnt_ref[...] = jnp.zeros_like(cnt_ref)

    xqT = xqT_ref[...]
    yT = yT_ref[...]
    pn = pn_ref[...]

    f = jax.lax.dot_general(pn, Wl_ref[...], (((1,), (1,)), ((), ())),
                            preferred_element_type=jnp.float32,
                            precision=jax.lax.Precision.HIGHEST) + bl_ref[...]

    d2 = jnp.zeros(o_ref.shape[1:2] + yT.shape[1:2], jnp.float32)
    for k in range(cd):
        diff = xq_ref[:, k:k + 1] - yT[k:k + 1, :]
        d2 = d2 + diff * diff
    mask = (d2 <= _R2).astype(jnp.float32)

    cnt_ref[...] += jnp.broadcast_to(jnp.sum(mask, axis=1, keepdims=True),
                                     cnt_ref.shape)

    a = jax.lax.dot_general(xqT, W1a_ref[...], (((0,), (0,)), ((), ())),
                            preferred_element_type=jnp.float32,
                            precision=jax.lax.Precision.HIGHEST) + b1_ref[...]
    bnT = jax.lax.dot_general(W1b_ref[...], yT, (((0,), (0,)), ((), ())),
                              preferred_element_type=jnp.float32,
                              precision=jax.lax.Precision.HIGHEST)

    acc = jnp.zeros_like(num_ref)
    for h in range(h_dim):
        g = jax.nn.gelu(a[:, h:h + 1] + bnT[h:h + 1, :]) * mask
        fh = f * W2_ref[h:h + 1, :]
        acc = acc + jax.lax.dot_general(g, fh, (((1,), (0,)), ((), ())),
                                        preferred_element_type=jnp.float32,
                                        precision=jax.lax.Precision.HIGHEST)
    acc = acc + jax.lax.dot_general(mask, f * b2_ref[...], (((1,), (0,)), ((), ())),
                                    preferred_element_type=jnp.float32,
                                    precision=jax.lax.Precision.HIGHEST)
    num_ref[...] += acc

    @pl.when(i == nblocks - 1)
    def _fin():
        o_ref[...] = (num_ref[...] / jnp.maximum(cnt_ref[...], 1.0))[None]


def _encode_one(y, pn, xq, W_lift, b_lift, W1, b1, W2, b2, *, nb):
    n, cd = y.shape
    p = xq.shape[0]
    cin = pn.shape[1]
    cout = W_lift.shape[0]
    h_dim = W1.shape[1]
    npad = ((n + nb - 1) // nb) * nb
    nblocks = npad // nb
    yT = jnp.pad(y.T, ((0, 0), (0, npad - n)), constant_values=3.0)
    pnp = jnp.pad(pn, ((0, npad - n), (0, 0)))

    body = functools.partial(_tc_body, nblocks=nblocks, h_dim=h_dim, cd=cd)
    out = pl.pallas_call(
        body,
        grid=(nblocks,),
        in_specs=[
            pl.BlockSpec((p, cd), lambda i: (0, 0)),
            pl.BlockSpec((cd, p), lambda i: (0, 0)),
            pl.BlockSpec((cd, nb), lambda i: (0, i)),
            pl.BlockSpec((nb, cin), lambda i: (i, 0)),
            pl.BlockSpec((cout, cin), lambda i: (0, 0)),
            pl.BlockSpec((1, cout), lambda i: (0, 0)),
            pl.BlockSpec((cd, h_dim), lambda i: (0, 0)),
            pl.BlockSpec((cd, h_dim), lambda i: (0, 0)),
            pl.BlockSpec((1, h_dim), lambda i: (0, 0)),
            pl.BlockSpec((h_dim, cout), lambda i: (0, 0)),
            pl.BlockSpec((1, cout), lambda i: (0, 0)),
        ],
        out_specs=pl.BlockSpec((1, p, cout), lambda i: (0, 0, 0)),
        out_shape=jax.ShapeDtypeStruct((1, p, cout), jnp.float32),
        scratch_shapes=[
            pltpu.VMEM((p, cout), jnp.float32),
            pltpu.VMEM((p, cout), jnp.float32),
        ],
    )(xq, xq.T, yT, pnp, W_lift, b_lift[None, :], W1[:cd], W1[cd:],
      b1[None, :], W2, b2[None, :])
    return out[0]


def kernel(x_coord, pndata, latent_tokens_coord, W_lift, b_lift, W1, b1, W2, b2):
    bsz = x_coord.shape[0]
    outs = [
        _encode_one(x_coord[b], pndata[b], latent_tokens_coord,
                    W_lift, b_lift, W1, b1, W2, b2, nb=1024)
        for b in range(bsz)
    ]
    return jnp.stack(outs, axis=0)


# speedup vs baseline: 6.9897x; 2.1748x over previous
import functools

import jax
import jax.numpy as jnp
from jax.experimental import pallas as pl
from jax.experimental.pallas import tpu as pltpu

_RADIUS = 0.09
_R2 = _RADIUS * _RADIUS


def _tc_body(xq_ref, xqT_ref, yT_ref, pn_ref, Wl_ref, bl_ref, W1a_ref, W1b_ref,
             b1_ref, W2_ref, b2_ref, o_ref, num_ref, cnt_ref, *, nblocks, h_dim,
             cd):
    i = pl.program_id(0)

    @pl.when(i == 0)
    def _init():
        num_ref[...] = jnp.zeros_like(num_ref)
        cnt_ref[...] = jnp.zeros_like(cnt_ref)

    xqT = xqT_ref[...]
    yT = yT_ref[...]
    pn = pn_ref[...]

    f = jax.lax.dot_general(pn, Wl_ref[...], (((1,), (1,)), ((), ())),
                            preferred_element_type=jnp.float32,
                            precision=jax.lax.Precision.HIGHEST) + bl_ref[...]

    d2 = jnp.zeros(o_ref.shape[1:2] + yT.shape[1:2], jnp.float32)
    for k in range(cd):
        diff = xq_ref[:, k:k + 1] - yT[k:k + 1, :]
        d2 = d2 + diff * diff
    mask = (d2 <= _R2).astype(jnp.float32)

    cnt_ref[...] += jnp.broadcast_to(jnp.sum(mask, axis=1, keepdims=True),
                                     cnt_ref.shape)

    a = jax.lax.dot_general(xqT, W1a_ref[...], (((0,), (0,)), ((), ())),
                            preferred_element_type=jnp.float32,
                            precision=jax.lax.Precision.HIGHEST) + b1_ref[...]
    bnT = jax.lax.dot_general(W1b_ref[...], yT, (((0,), (0,)), ((), ())),
                              preferred_element_type=jnp.float32,
                              precision=jax.lax.Precision.HIGHEST)

    acc = jnp.zeros_like(num_ref)
    for h in range(h_dim):
        g = jax.nn.gelu(a[:, h:h + 1] + bnT[h:h + 1, :]) * mask
        fh = f * W2_ref[h:h + 1, :]
        acc = acc + jax.lax.dot_general(g, fh, (((1,), (0,)), ((), ())),
                                        preferred_element_type=jnp.float32)
    acc = acc + jax.lax.dot_general(mask, f * b2_ref[...], (((1,), (0,)), ((), ())),
                                    preferred_element_type=jnp.float32)
    num_ref[...] += acc

    @pl.when(i == nblocks - 1)
    def _fin():
        o_ref[...] = (num_ref[...] / jnp.maximum(cnt_ref[...], 1.0))[None]


def _encode_one(y, pn, xq, W_lift, b_lift, W1, b1, W2, b2, *, nb):
    n, cd = y.shape
    p = xq.shape[0]
    cin = pn.shape[1]
    cout = W_lift.shape[0]
    h_dim = W1.shape[1]
    npad = ((n + nb - 1) // nb) * nb
    nblocks = npad // nb
    yT = jnp.pad(y.T, ((0, 0), (0, npad - n)), constant_values=3.0)
    pnp = jnp.pad(pn, ((0, npad - n), (0, 0)))

    body = functools.partial(_tc_body, nblocks=nblocks, h_dim=h_dim, cd=cd)
    out = pl.pallas_call(
        body,
        grid=(nblocks,),
        in_specs=[
            pl.BlockSpec((p, cd), lambda i: (0, 0)),
            pl.BlockSpec((cd, p), lambda i: (0, 0)),
            pl.BlockSpec((cd, nb), lambda i: (0, i)),
            pl.BlockSpec((nb, cin), lambda i: (i, 0)),
            pl.BlockSpec((cout, cin), lambda i: (0, 0)),
            pl.BlockSpec((1, cout), lambda i: (0, 0)),
            pl.BlockSpec((cd, h_dim), lambda i: (0, 0)),
            pl.BlockSpec((cd, h_dim), lambda i: (0, 0)),
            pl.BlockSpec((1, h_dim), lambda i: (0, 0)),
            pl.BlockSpec((h_dim, cout), lambda i: (0, 0)),
            pl.BlockSpec((1, cout), lambda i: (0, 0)),
        ],
        out_specs=pl.BlockSpec((1, p, cout), lambda i: (0, 0, 0)),
        out_shape=jax.ShapeDtypeStruct((1, p, cout), jnp.float32),
        scratch_shapes=[
            pltpu.VMEM((p, cout), jnp.float32),
            pltpu.VMEM((p, cout), jnp.float32),
        ],
    )(xq, xq.T, yT, pnp, W_lift, b_lift[None, :], W1[:cd], W1[cd:],
      b1[None, :], W2, b2[None, :])
    return out[0]


def kernel(x_coord, pndata, latent_tokens_coord, W_lift, b_lift, W1, b1, W2, b2):
    bsz = x_coord.shape[0]
    outs = [
        _encode_one(x_coord[b], pndata[b], latent_tokens_coord,
                    W_lift, b_lift, W1, b1, W2, b2, nb=1024)
        for b in range(bsz)
    ]
    return jnp.stack(outs, axis=0)


# speedup vs baseline: 7.0785x; 1.0127x over previous
import functools

import jax
import jax.numpy as jnp
from jax.experimental import pallas as pl
from jax.experimental.pallas import tpu as pltpu

_RADIUS = 0.09
_R2 = _RADIUS * _RADIUS


def _tc_body(xq_ref, xqT_ref, yT_ref, pn_ref, Wl_ref, bl_ref, W1a_ref, W1b_ref,
             b1_ref, W2_ref, b2_ref, o_ref, num_ref, cnt_ref, *, nblocks, h_dim,
             cd):
    i = pl.program_id(0)

    @pl.when(i == 0)
    def _init():
        num_ref[...] = jnp.zeros_like(num_ref)
        cnt_ref[...] = jnp.zeros_like(cnt_ref)

    xqT = xqT_ref[...]
    yT = yT_ref[...]
    pn = pn_ref[...]

    f = jax.lax.dot_general(pn, Wl_ref[...], (((1,), (1,)), ((), ())),
                            preferred_element_type=jnp.float32) + bl_ref[...]

    d2 = jnp.zeros(o_ref.shape[1:2] + yT.shape[1:2], jnp.float32)
    for k in range(cd):
        diff = xq_ref[:, k:k + 1] - yT[k:k + 1, :]
        d2 = d2 + diff * diff
    mask = (d2 <= _R2).astype(jnp.float32)

    cnt_ref[...] += jnp.broadcast_to(jnp.sum(mask, axis=1, keepdims=True),
                                     cnt_ref.shape)

    a = jax.lax.dot_general(xqT, W1a_ref[...], (((0,), (0,)), ((), ())),
                            preferred_element_type=jnp.float32,
                            precision=jax.lax.Precision.HIGHEST) + b1_ref[...]
    bnT = jax.lax.dot_general(W1b_ref[...], yT, (((0,), (0,)), ((), ())),
                              preferred_element_type=jnp.float32,
                              precision=jax.lax.Precision.HIGHEST)

    acc = jnp.zeros_like(num_ref)
    for h in range(h_dim):
        g = jax.nn.gelu(a[:, h:h + 1] + bnT[h:h + 1, :]) * mask
        fh = f * W2_ref[h:h + 1, :]
        acc = acc + jax.lax.dot_general(g, fh, (((1,), (0,)), ((), ())),
                                        preferred_element_type=jnp.float32)
    acc = acc + jax.lax.dot_general(mask, f * b2_ref[...], (((1,), (0,)), ((), ())),
                                    preferred_element_type=jnp.float32)
    num_ref[...] += acc

    @pl.when(i == nblocks - 1)
    def _fin():
        o_ref[...] = (num_ref[...] / jnp.maximum(cnt_ref[...], 1.0))[None]


def _encode_one(y, pn, xq, W_lift, b_lift, W1, b1, W2, b2, *, nb):
    n, cd = y.shape
    p = xq.shape[0]
    cin = pn.shape[1]
    cout = W_lift.shape[0]
    h_dim = W1.shape[1]
    npad = ((n + nb - 1) // nb) * nb
    nblocks = npad // nb
    yT = jnp.pad(y.T, ((0, 0), (0, npad - n)), constant_values=3.0)
    pnp = jnp.pad(pn, ((0, npad - n), (0, 0)))

    body = functools.partial(_tc_body, nblocks=nblocks, h_dim=h_dim, cd=cd)
    out = pl.pallas_call(
        body,
        grid=(nblocks,),
        in_specs=[
            pl.BlockSpec((p, cd), lambda i: (0, 0)),
            pl.BlockSpec((cd, p), lambda i: (0, 0)),
            pl.BlockSpec((cd, nb), lambda i: (0, i)),
            pl.BlockSpec((nb, cin), lambda i: (i, 0)),
            pl.BlockSpec((cout, cin), lambda i: (0, 0)),
            pl.BlockSpec((1, cout), lambda i: (0, 0)),
            pl.BlockSpec((cd, h_dim), lambda i: (0, 0)),
            pl.BlockSpec((cd, h_dim), lambda i: (0, 0)),
            pl.BlockSpec((1, h_dim), lambda i: (0, 0)),
            pl.BlockSpec((h_dim, cout), lambda i: (0, 0)),
            pl.BlockSpec((1, cout), lambda i: (0, 0)),
        ],
        out_specs=pl.BlockSpec((1, p, cout), lambda i: (0, 0, 0)),
        out_shape=jax.ShapeDtypeStruct((1, p, cout), jnp.float32),
        scratch_shapes=[
            pltpu.VMEM((p, cout), jnp.float32),
            pltpu.VMEM((p, cout), jnp.float32),
        ],
    )(xq, xq.T, yT, pnp, W_lift, b_lift[None, :], W1[:cd], W1[cd:],
      b1[None, :], W2, b2[None, :])
    return out[0]


def kernel(x_coord, pndata, latent_tokens_coord, W_lift, b_lift, W1, b1, W2, b2):
    bsz = x_coord.shape[0]
    outs = [
        _encode_one(x_coord[b], pndata[b], latent_tokens_coord,
                    W_lift, b_lift, W1, b1, W2, b2, nb=1024)
        for b in range(bsz)
    ]
    return jnp.stack(outs, axis=0)
